# baseline (speedup 1.0000x reference)
import functools

import jax
import jax.numpy as jnp
from jax import lax
from jax.experimental import pallas as pl
from jax.experimental.pallas import tpu as pltpu
from jax.experimental.pallas import tpu_sc as plsc

N_ROWS = 100000
D = 64
NUM_CORES = 2
NUM_SUBCORES = 16
NUM_WORKERS = NUM_CORES * NUM_SUBCORES

CHUNK = 200
NCHUNK = N_ROWS // CHUNK
MAX_CHUNKS_PER_WORKER = -(-NCHUNK // NUM_WORKERS)


def _gather_body(table_hbm, idx_hbm, out_hbm, idx_v, rows_v, sem):
    wid = lax.axis_index("s") * NUM_CORES + lax.axis_index("c")
    for it in range(MAX_CHUNKS_PER_WORKER):
        c = wid + it * NUM_WORKERS

        @pl.when(c < NCHUNK)
        def _():
            base = c * CHUNK
            pltpu.sync_copy(idx_hbm.at[pl.ds(base, CHUNK)], idx_v)
            pltpu.async_copy(table_hbm.at[idx_v], rows_v, sem).wait()
            pltpu.sync_copy(rows_v, out_hbm.at[pl.ds(base, CHUNK)])


@jax.jit
def _embed(table, ent_idx):
    mesh = plsc.VectorSubcoreMesh(core_axis_name="c", subcore_axis_name="s")
    out = pl.kernel(
        _gather_body,
        mesh=mesh,
        out_type=jax.ShapeDtypeStruct((N_ROWS, D), jnp.float32),
        scratch_types=[
            pltpu.VMEM((CHUNK,), jnp.int32),
            pltpu.VMEM((CHUNK, D), jnp.float32),
            pltpu.SemaphoreType.DMA,
        ],
        compiler_params=pltpu.CompilerParams(use_tc_tiling_on_sc=False),
    )(table, ent_idx)
    return out[None, None, :, :]


def kernel(table, ent_idx):
    return _embed(table, ent_idx.astype(jnp.int32))


# speedup vs baseline: 1.1191x; 1.1191x over previous
import jax
import jax.numpy as jnp
from jax import lax
from jax.experimental import pallas as pl
from jax.experimental.pallas import tpu as pltpu
from jax.experimental.pallas import tpu_sc as plsc

N_ROWS = 100000
D = 64
NUM_CORES = 2
NUM_SUBCORES = 16
NUM_WORKERS = NUM_CORES * NUM_SUBCORES

CHUNK = 800
NCHUNK = N_ROWS // CHUNK
MAX_K = -(-NCHUNK // NUM_WORKERS)
NBUF = 2


def _gather_body(table_hbm, idx_hbm, out_hbm, idx_v, rows_v, isem, gsem, ssem):
    wid = lax.axis_index("s") * NUM_CORES + lax.axis_index("c")

    def chunk_id(k):
        c = wid + k * NUM_WORKERS
        return jnp.where(c < NCHUNK, c, wid)

    bases = [chunk_id(k) * CHUNK for k in range(MAX_K)]

    idx_copies = [
        pltpu.async_copy(idx_hbm.at[pl.ds(bases[k], CHUNK)], idx_v.at[k], isem)
        for k in range(MAX_K)
    ]
    for cp in idx_copies:
        cp.wait()

    def start_gather(k):
        return pltpu.async_copy(
            table_hbm.at[idx_v.at[k]], rows_v.at[k % NBUF], gsem
        )

    def start_scatter(k):
        return pltpu.async_copy(
            rows_v.at[k % NBUF], out_hbm.at[pl.ds(bases[k], CHUNK)], ssem
        )

    gathers = [None] * MAX_K
    scatters = [None] * MAX_K
    gathers[0] = start_gather(0)
    if MAX_K > 1:
        gathers[1] = start_gather(1)
    gathers[0].wait()
    scatters[0] = start_scatter(0)
    for k in range(2, MAX_K):
        scatters[k - 2].wait()
        gathers[k] = start_gather(k)
        gathers[k - 1].wait()
        scatters[k - 1] = start_scatter(k - 1)
    if MAX_K > 1:
        if MAX_K > 2:
            scatters[MAX_K - 2].wait()
        gathers[MAX_K - 1].wait()
        scatters[MAX_K - 1] = start_scatter(MAX_K - 1)
    scatters[MAX_K - 1].wait()


@jax.jit
def _embed(table, ent_idx):
    mesh = plsc.VectorSubcoreMesh(core_axis_name="c", subcore_axis_name="s")
    out = pl.kernel(
        _gather_body,
        mesh=mesh,
        out_type=jax.ShapeDtypeStruct((N_ROWS, D), jnp.float32),
        scratch_types=[
            pltpu.VMEM((MAX_K, CHUNK), jnp.int32),
            pltpu.VMEM((NBUF, CHUNK, D), jnp.float32),
            pltpu.SemaphoreType.DMA,
            pltpu.SemaphoreType.DMA,
            pltpu.SemaphoreType.DMA,
        ],
        compiler_params=pltpu.CompilerParams(use_tc_tiling_on_sc=False),
    )(table, ent_idx)
    return out[None, None, :, :]


def kernel(table, ent_idx):
    return _embed(table, ent_idx.astype(jnp.int32))
